# Initial kernel scaffold; baseline (speedup 1.0000x reference)
import functools

import jax
import jax.numpy as jnp
from jax import lax
from jax.experimental import pallas as pl
from jax.experimental.pallas import tpu as pltpu
from jax.experimental.pallas import tpu_sc as plsc

N = 10000
L = 104
LP = 112
E = 640000
K = 128
NSUB = 16
ROWS_PER_SUB = 313
ROWS = NSUB * ROWS_PER_SUB
EPAD = ROWS * K
NACC = N + 16
ACC_PER_SUB = NACC // NSUB
NBLK = 10
BLK = N // NBLK


def _sc_segsum(qh, qp, src2d, dst2d, znull):
    mesh = plsc.VectorSubcoreMesh(core_axis_name="c", subcore_axis_name="s")

    @functools.partial(
        pl.kernel,
        mesh=mesh,
        out_type=(
            jax.ShapeDtypeStruct((NACC, LP), jnp.float32),
            jax.ShapeDtypeStruct((NACC, LP), jnp.float32),
        ),
        scratch_types=[
            pltpu.VMEM((K,), jnp.int32),
            pltpu.VMEM((K,), jnp.int32),
            pltpu.VMEM((K, LP), jnp.float32),
            pltpu.VMEM_SHARED((NACC, LP), jnp.float32),
            pltpu.SemaphoreType.DMA,
        ],
    )
    def sc_kernel(qh_hbm, qp_hbm, src_hbm, dst_hbm, z_hbm, onum, oden,
                  sidx, didx, rows, acc, sem):
        core = lax.axis_index("c")
        sub = lax.axis_index("s")

        a0 = sub * ACC_PER_SUB
        pltpu.sync_copy(z_hbm.at[pl.ds(a0, ACC_PER_SUB)],
                        acc.at[pl.ds(a0, ACC_PER_SUB)])
        plsc.subcore_barrier()

        def edge_pass(table_hbm):
            r0 = sub * ROWS_PER_SUB

            @pl.loop(0, ROWS_PER_SUB)
            def _(g):
                r = r0 + g
                pltpu.sync_copy(src_hbm.at[r], sidx)
                pltpu.sync_copy(dst_hbm.at[r], didx)
                pltpu.async_copy(table_hbm.at[sidx], rows, sem).wait()
                pltpu.sync_copy(rows, acc.at[didx], add=True)

        @pl.when(core == 0)
        def _():
            edge_pass(qh_hbm)

        @pl.when(core == 1)
        def _():
            edge_pass(qp_hbm)

        plsc.subcore_barrier()

        @pl.when(core == 0)
        def _():
            pltpu.sync_copy(acc.at[pl.ds(a0, ACC_PER_SUB)],
                            onum.at[pl.ds(a0, ACC_PER_SUB)])

        @pl.when(core == 1)
        def _():
            pltpu.sync_copy(acc.at[pl.ds(a0, ACC_PER_SUB)],
                            oden.at[pl.ds(a0, ACC_PER_SUB)])

    return sc_kernel(qh, qp, src2d, dst2d, znull)


def _h0_body(x_ref, w1_ref, b1_ref, attw_ref, h0_ref, c_ref):
    h0 = jnp.maximum(
        jnp.dot(x_ref[...], w1_ref[...], preferred_element_type=jnp.float32)
        + b1_ref[...], 0.0)
    h0_ref[...] = h0
    m = jnp.max(h0 * attw_ref[...], axis=0, keepdims=True)

    @pl.when(pl.program_id(0) == 0)
    def _():
        c_ref[...] = m

    @pl.when(pl.program_id(0) > 0)
    def _():
        c_ref[...] = jnp.maximum(c_ref[...], m)


def _call_h0(x, W1, b1r, attwr):
    return pl.pallas_call(
        _h0_body,
        grid=(NBLK,),
        in_specs=[
            pl.BlockSpec((BLK, L), lambda i: (i, 0)),
            pl.BlockSpec((L, L), lambda i: (0, 0)),
            pl.BlockSpec((1, L), lambda i: (0, 0)),
            pl.BlockSpec((1, L), lambda i: (0, 0)),
        ],
        out_specs=[
            pl.BlockSpec((BLK, L), lambda i: (i, 0)),
            pl.BlockSpec((1, L), lambda i: (0, 0)),
        ],
        out_shape=[
            jax.ShapeDtypeStruct((N, L), jnp.float32),
            jax.ShapeDtypeStruct((1, L), jnp.float32),
        ],
    )(x, W1, b1r, attwr)


def _q_body(h_ref, attw_ref, c_ref, qh_ref, qp_ref):
    p = jnp.exp(h_ref[...] * attw_ref[...] - c_ref[...])
    z = jnp.zeros((BLK, LP - L), jnp.float32)
    qh_ref[...] = jnp.concatenate([p * h_ref[...], z], axis=1)
    qp_ref[...] = jnp.concatenate([p, z], axis=1)


def _call_q(h, attwr, c):
    return pl.pallas_call(
        _q_body,
        grid=(NBLK,),
        in_specs=[
            pl.BlockSpec((BLK, L), lambda i: (i, 0)),
            pl.BlockSpec((1, L), lambda i: (0, 0)),
            pl.BlockSpec((1, L), lambda i: (0, 0)),
        ],
        out_specs=[
            pl.BlockSpec((BLK, LP), lambda i: (i, 0)),
            pl.BlockSpec((BLK, LP), lambda i: (i, 0)),
        ],
        out_shape=[
            jax.ShapeDtypeStruct((N, LP), jnp.float32),
            jax.ShapeDtypeStruct((N, LP), jnp.float32),
        ],
    )(h, attwr, c)


def _merge_body(num_ref, den_ref, x_ref, h0_ref, wmt_ref, wmb_ref, bm_ref,
                wc_ref, h_ref, c_ref):
    den = den_ref[:, :L]
    agg = num_ref[:, :L] / (den + 1e-9)
    merged = jnp.maximum(
        jnp.dot(agg, wmt_ref[...], preferred_element_type=jnp.float32)
        + jnp.dot(x_ref[...], wmb_ref[...], preferred_element_type=jnp.float32)
        + bm_ref[...], 0.0)
    h = jnp.where(den[:, :1] > 0.0, merged, h0_ref[...])
    h_ref[...] = h
    m = jnp.max(h * wc_ref[...], axis=0, keepdims=True)

    @pl.when(pl.program_id(0) == 0)
    def _():
        c_ref[...] = m

    @pl.when(pl.program_id(0) > 0)
    def _():
        c_ref[...] = jnp.maximum(c_ref[...], m)


def _call_merge(num, den, x, h0, WmT, WmB, bmr, wcr):
    return pl.pallas_call(
        _merge_body,
        grid=(NBLK,),
        in_specs=[
            pl.BlockSpec((BLK, LP), lambda i: (i, 0)),
            pl.BlockSpec((BLK, LP), lambda i: (i, 0)),
            pl.BlockSpec((BLK, L), lambda i: (i, 0)),
            pl.BlockSpec((BLK, L), lambda i: (i, 0)),
            pl.BlockSpec((L, L), lambda i: (0, 0)),
            pl.BlockSpec((L, L), lambda i: (0, 0)),
            pl.BlockSpec((1, L), lambda i: (0, 0)),
            pl.BlockSpec((1, L), lambda i: (0, 0)),
        ],
        out_specs=[
            pl.BlockSpec((BLK, L), lambda i: (i, 0)),
            pl.BlockSpec((1, L), lambda i: (0, 0)),
        ],
        out_shape=[
            jax.ShapeDtypeStruct((N, L), jnp.float32),
            jax.ShapeDtypeStruct((1, L), jnp.float32),
        ],
    )(num, den, x, h0, WmT, WmB, bmr, wcr)


def _final1_body(h_ref, dagw_ref, c_ref, se_ref, seh_ref):
    h = h_ref[...]
    e = jnp.exp(h * dagw_ref[...] - c_ref[...])
    se = jnp.sum(e, axis=0, keepdims=True)
    seh = jnp.sum(e * h, axis=0, keepdims=True)

    @pl.when(pl.program_id(0) == 0)
    def _():
        se_ref[...] = se
        seh_ref[...] = seh

    @pl.when(pl.program_id(0) > 0)
    def _():
        se_ref[...] = se_ref[...] + se
        seh_ref[...] = seh_ref[...] + seh


def _call_final1(h, dagwr, cdag):
    return pl.pallas_call(
        _final1_body,
        grid=(NBLK,),
        in_specs=[
            pl.BlockSpec((BLK, L), lambda i: (i, 0)),
            pl.BlockSpec((1, L), lambda i: (0, 0)),
            pl.BlockSpec((1, L), lambda i: (0, 0)),
        ],
        out_specs=[
            pl.BlockSpec((1, L), lambda i: (0, 0)),
            pl.BlockSpec((1, L), lambda i: (0, 0)),
        ],
        out_shape=[
            jax.ShapeDtypeStruct((1, L), jnp.float32),
            jax.ShapeDtypeStruct((1, L), jnp.float32),
        ],
    )(h, dagwr, cdag)


def _final2_body(se_ref, seh_ref, wf_ref, bf_ref, out_ref):
    pooled = seh_ref[...] / se_ref[...]
    out_ref[...] = (
        jnp.dot(pooled, wf_ref[...], preferred_element_type=jnp.float32)
        + bf_ref[...])


def _call_final2(se, seh, Wf, bfr):
    nc = Wf.shape[1]
    return pl.pallas_call(
        _final2_body,
        in_specs=[
            pl.BlockSpec((1, L), lambda: (0, 0)),
            pl.BlockSpec((1, L), lambda: (0, 0)),
            pl.BlockSpec((L, nc), lambda: (0, 0)),
            pl.BlockSpec((1, nc), lambda: (0, 0)),
        ],
        out_specs=pl.BlockSpec((1, nc), lambda: (0, 0)),
        out_shape=jax.ShapeDtypeStruct((1, nc), jnp.float32),
    )(se, seh, Wf, bfr)


def kernel(x, edge_index, W1, b1, Wm, bm, att_w, dag_w, Wf, bf):
    src = edge_index[0]
    dst = edge_index[1]
    pad = EPAD - E
    src2d = jnp.concatenate([src, jnp.zeros((pad,), jnp.int32)]).reshape(ROWS, K)
    dst2d = jnp.concatenate([dst, jnp.full((pad,), N, jnp.int32)]).reshape(ROWS, K)
    znull = jnp.zeros((NACC, LP), jnp.float32)

    b1r = b1.reshape(1, L)
    bmr = bm.reshape(1, L)
    bfr = bf.reshape(1, -1)
    attwr = att_w.reshape(1, L)
    dagwr = dag_w.reshape(1, L)
    WmT = Wm[:L]
    WmB = Wm[L:]

    h0, c = _call_h0(x, W1, b1r, attwr)

    h = h0
    for step in range(2):
        qh, qp = _call_q(h, attwr, c)
        num, den = _sc_segsum(qh, qp, src2d, dst2d, znull)
        wcr = attwr if step == 0 else dagwr
        h, c = _call_merge(num[:N], den[:N], x, h0, WmT, WmB, bmr, wcr)

    se, seh = _call_final1(h, dagwr, c)
    out = _call_final2(se, seh, Wf, bfr)
    return out.reshape(-1)


# speedup vs baseline: 10.0895x; 10.0895x over previous
import functools

import jax
import jax.numpy as jnp
from jax import lax
from jax.experimental import pallas as pl
from jax.experimental.pallas import tpu as pltpu
from jax.experimental.pallas import tpu_sc as plsc

N = 10000
L = 104
LP = 128
E = 640000
K = 128
NSUB = 16
ROWS_PER_SUB = 313
ROWS = NSUB * ROWS_PER_SUB
EPAD = ROWS * K
ACC_PER_SUB = 632
NACC = NSUB * ACC_PER_SUB
NBLK = 10
BLK = N // NBLK


def _sc_segsum(qh, qp, src1d, dst1d, znull):
    mesh = plsc.VectorSubcoreMesh(core_axis_name="c", subcore_axis_name="s")

    @functools.partial(
        pl.kernel,
        mesh=mesh,
        out_type=(
            jax.ShapeDtypeStruct((NACC, LP), jnp.float32),
            jax.ShapeDtypeStruct((NACC, LP), jnp.float32),
        ),
        scratch_types=[
            pltpu.VMEM((K,), jnp.int32),
            pltpu.VMEM((K,), jnp.int32),
            pltpu.VMEM((K, LP), jnp.float32),
            pltpu.VMEM_SHARED((NACC, LP), jnp.float32),
            pltpu.SemaphoreType.DMA,
        ],
    )
    def sc_kernel(qh_hbm, qp_hbm, src_hbm, dst_hbm, z_hbm, onum, oden,
                  sidx, didx, rows, acc, sem):
        core = lax.axis_index("c")
        sub = lax.axis_index("s")

        a0 = sub * ACC_PER_SUB
        pltpu.sync_copy(z_hbm.at[pl.ds(a0, ACC_PER_SUB)],
                        acc.at[pl.ds(a0, ACC_PER_SUB)])
        plsc.subcore_barrier()

        def edge_pass(table_hbm):
            r0 = sub * ROWS_PER_SUB

            @pl.loop(0, ROWS_PER_SUB)
            def _(g):
                r = r0 + g
                pltpu.sync_copy(src_hbm.at[pl.ds(r * K, K)], sidx)
                pltpu.sync_copy(dst_hbm.at[pl.ds(r * K, K)], didx)
                pltpu.async_copy(table_hbm.at[sidx], rows, sem).wait()
                pltpu.sync_copy(rows, acc.at[didx], add=True)

        @pl.when(core == 0)
        def _():
            edge_pass(qh_hbm)

        @pl.when(core == 1)
        def _():
            edge_pass(qp_hbm)

        plsc.subcore_barrier()

        @pl.when(core == 0)
        def _():
            pltpu.sync_copy(acc.at[pl.ds(a0, ACC_PER_SUB)],
                            onum.at[pl.ds(a0, ACC_PER_SUB)])

        @pl.when(core == 1)
        def _():
            pltpu.sync_copy(acc.at[pl.ds(a0, ACC_PER_SUB)],
                            oden.at[pl.ds(a0, ACC_PER_SUB)])

    return sc_kernel(qh, qp, src1d, dst1d, znull)


def _h0_body(x_ref, w1_ref, b1_ref, attw_ref, h0_ref, c_ref):
    h0 = jnp.maximum(
        jnp.dot(x_ref[...], w1_ref[...], preferred_element_type=jnp.float32)
        + b1_ref[...], 0.0)
    h0_ref[...] = h0
    m = jnp.max(h0 * attw_ref[...], axis=0, keepdims=True)

    @pl.when(pl.program_id(0) == 0)
    def _():
        c_ref[...] = m

    @pl.when(pl.program_id(0) > 0)
    def _():
        c_ref[...] = jnp.maximum(c_ref[...], m)


def _call_h0(x, W1, b1r, attwr):
    return pl.pallas_call(
        _h0_body,
        grid=(NBLK,),
        in_specs=[
            pl.BlockSpec((BLK, L), lambda i: (i, 0)),
            pl.BlockSpec((L, L), lambda i: (0, 0)),
            pl.BlockSpec((1, L), lambda i: (0, 0)),
            pl.BlockSpec((1, L), lambda i: (0, 0)),
        ],
        out_specs=[
            pl.BlockSpec((BLK, L), lambda i: (i, 0)),
            pl.BlockSpec((1, L), lambda i: (0, 0)),
        ],
        out_shape=[
            jax.ShapeDtypeStruct((N, L), jnp.float32),
            jax.ShapeDtypeStruct((1, L), jnp.float32),
        ],
    )(x, W1, b1r, attwr)


def _q_body(h_ref, attw_ref, c_ref, qh_ref, qp_ref):
    p = jnp.exp(h_ref[...] * attw_ref[...] - c_ref[...])
    z = jnp.zeros((BLK, LP - L), jnp.float32)
    qh_ref[...] = jnp.concatenate([p * h_ref[...], z], axis=1)
    qp_ref[...] = jnp.concatenate([p, z], axis=1)


def _call_q(h, attwr, c):
    return pl.pallas_call(
        _q_body,
        grid=(NBLK,),
        in_specs=[
            pl.BlockSpec((BLK, L), lambda i: (i, 0)),
            pl.BlockSpec((1, L), lambda i: (0, 0)),
            pl.BlockSpec((1, L), lambda i: (0, 0)),
        ],
        out_specs=[
            pl.BlockSpec((BLK, LP), lambda i: (i, 0)),
            pl.BlockSpec((BLK, LP), lambda i: (i, 0)),
        ],
        out_shape=[
            jax.ShapeDtypeStruct((N, LP), jnp.float32),
            jax.ShapeDtypeStruct((N, LP), jnp.float32),
        ],
    )(h, attwr, c)


def _merge_body(num_ref, den_ref, x_ref, h0_ref, wmt_ref, wmb_ref, bm_ref,
                wc_ref, h_ref, c_ref):
    den = den_ref[:, :L]
    agg = num_ref[:, :L] / (den + 1e-9)
    merged = jnp.maximum(
        jnp.dot(agg, wmt_ref[...], preferred_element_type=jnp.float32)
        + jnp.dot(x_ref[...], wmb_ref[...], preferred_element_type=jnp.float32)
        + bm_ref[...], 0.0)
    h = jnp.where(den[:, :1] > 0.0, merged, h0_ref[...])
    h_ref[...] = h
    m = jnp.max(h * wc_ref[...], axis=0, keepdims=True)

    @pl.when(pl.program_id(0) == 0)
    def _():
        c_ref[...] = m

    @pl.when(pl.program_id(0) > 0)
    def _():
        c_ref[...] = jnp.maximum(c_ref[...], m)


def _call_merge(num, den, x, h0, WmT, WmB, bmr, wcr):
    return pl.pallas_call(
        _merge_body,
        grid=(NBLK,),
        in_specs=[
            pl.BlockSpec((BLK, LP), lambda i: (i, 0)),
            pl.BlockSpec((BLK, LP), lambda i: (i, 0)),
            pl.BlockSpec((BLK, L), lambda i: (i, 0)),
            pl.BlockSpec((BLK, L), lambda i: (i, 0)),
            pl.BlockSpec((L, L), lambda i: (0, 0)),
            pl.BlockSpec((L, L), lambda i: (0, 0)),
            pl.BlockSpec((1, L), lambda i: (0, 0)),
            pl.BlockSpec((1, L), lambda i: (0, 0)),
        ],
        out_specs=[
            pl.BlockSpec((BLK, L), lambda i: (i, 0)),
            pl.BlockSpec((1, L), lambda i: (0, 0)),
        ],
        out_shape=[
            jax.ShapeDtypeStruct((N, L), jnp.float32),
            jax.ShapeDtypeStruct((1, L), jnp.float32),
        ],
    )(num, den, x, h0, WmT, WmB, bmr, wcr)


def _final1_body(h_ref, dagw_ref, c_ref, se_ref, seh_ref):
    h = h_ref[...]
    e = jnp.exp(h * dagw_ref[...] - c_ref[...])
    se = jnp.sum(e, axis=0, keepdims=True)
    seh = jnp.sum(e * h, axis=0, keepdims=True)

    @pl.when(pl.program_id(0) == 0)
    def _():
        se_ref[...] = se
        seh_ref[...] = seh

    @pl.when(pl.program_id(0) > 0)
    def _():
        se_ref[...] = se_ref[...] + se
        seh_ref[...] = seh_ref[...] + seh


def _call_final1(h, dagwr, cdag):
    return pl.pallas_call(
        _final1_body,
        grid=(NBLK,),
        in_specs=[
            pl.BlockSpec((BLK, L), lambda i: (i, 0)),
            pl.BlockSpec((1, L), lambda i: (0, 0)),
            pl.BlockSpec((1, L), lambda i: (0, 0)),
        ],
        out_specs=[
            pl.BlockSpec((1, L), lambda i: (0, 0)),
            pl.BlockSpec((1, L), lambda i: (0, 0)),
        ],
        out_shape=[
            jax.ShapeDtypeStruct((1, L), jnp.float32),
            jax.ShapeDtypeStruct((1, L), jnp.float32),
        ],
    )(h, dagwr, cdag)


def _final2_body(se_ref, seh_ref, wf_ref, bf_ref, out_ref):
    pooled = seh_ref[...] / se_ref[...]
    out_ref[...] = (
        jnp.dot(pooled, wf_ref[...], preferred_element_type=jnp.float32)
        + bf_ref[...])


def _call_final2(se, seh, Wf, bfr):
    nc = Wf.shape[1]
    return pl.pallas_call(
        _final2_body,
        in_specs=[
            pl.BlockSpec((1, L), lambda: (0, 0)),
            pl.BlockSpec((1, L), lambda: (0, 0)),
            pl.BlockSpec((L, nc), lambda: (0, 0)),
            pl.BlockSpec((1, nc), lambda: (0, 0)),
        ],
        out_specs=pl.BlockSpec((1, nc), lambda: (0, 0)),
        out_shape=jax.ShapeDtypeStruct((1, nc), jnp.float32),
    )(se, seh, Wf, bfr)


def kernel(x, edge_index, W1, b1, Wm, bm, att_w, dag_w, Wf, bf):
    src = edge_index[0]
    dst = edge_index[1]
    pad = EPAD - E
    src1d = jnp.concatenate([src, jnp.zeros((pad,), jnp.int32)])
    dst1d = jnp.concatenate([dst, jnp.full((pad,), N, jnp.int32)])
    znull = jnp.zeros((NACC, LP), jnp.float32)

    b1r = b1.reshape(1, L)
    bmr = bm.reshape(1, L)
    bfr = bf.reshape(1, -1)
    attwr = att_w.reshape(1, L)
    dagwr = dag_w.reshape(1, L)
    WmT = Wm[:L]
    WmB = Wm[L:]

    h0, c = _call_h0(x, W1, b1r, attwr)

    h = h0
    for step in range(2):
        qh, qp = _call_q(h, attwr, c)
        num, den = _sc_segsum(qh, qp, src1d, dst1d, znull)
        wcr = attwr if step == 0 else dagwr
        h, c = _call_merge(num[:N], den[:N], x, h0, WmT, WmB, bmr, wcr)

    se, seh = _call_final1(h, dagwr, c)
    out = _call_final2(se, seh, Wf, bfr)
    return out.reshape(-1)
